# baseline (speedup 1.0000x reference)
import jax
import jax.numpy as jnp
from jax import lax
from jax.experimental import pallas as pl
from jax.experimental.pallas import tpu as pltpu
from jax.experimental.pallas import tpu_sc as plsc

def _dot16(a, b):
    return jnp.dot(a.astype(jnp.bfloat16), b.astype(jnp.bfloat16),
                   preferred_element_type=jnp.float32)


_G, _NPG, _N, _E = 10, 1000, 10000, 160000
_DH = 32
_NC, _NS = 2, 16
_NW = _NC * _NS
_RPT = 128
_ER = _E // _RPT
_ER_BASE = _ER // _NW
_ER_EXTRA = _ER - _ER_BASE * _NW


def _proj_body(lx_ref, feat_ref, emb_ref, lw_ref, lb_ref, rw_ref, rb_ref,
               lout_ref, rout_ref):
    lout_ref[...] = jax.nn.relu(_dot16(lx_ref[...], lw_ref[...]) + lb_ref[...])
    emb_proj = _dot16(emb_ref[...], rw_ref[...])
    feat = feat_ref[...]
    onehot = (lax.broadcasted_iota(jnp.int32, (feat.shape[0], 32), 1)
              == feat).astype(jnp.float32)
    rout_ref[...] = jax.nn.relu(
        jnp.dot(onehot, emb_proj, preferred_element_type=jnp.float32) + rb_ref[...])


def _run_proj(lig_x, feat, emb_pad, lw, lb, rw, rb):
    blk = 2000
    nb = _N // blk
    return pl.pallas_call(
        _proj_body,
        grid=(nb,),
        in_specs=[
            pl.BlockSpec((blk, 128), lambda i: (i, 0)),
            pl.BlockSpec((blk, 1), lambda i: (i, 0)),
            pl.BlockSpec((32, 64), lambda i: (0, 0)),
            pl.BlockSpec((128, 32), lambda i: (0, 0)),
            pl.BlockSpec((1, 32), lambda i: (0, 0)),
            pl.BlockSpec((64, 32), lambda i: (0, 0)),
            pl.BlockSpec((1, 32), lambda i: (0, 0)),
        ],
        out_specs=[pl.BlockSpec((blk, 32), lambda i: (i, 0)),
                   pl.BlockSpec((blk, 32), lambda i: (i, 0))],
        out_shape=[jax.ShapeDtypeStruct((_N, 32), jnp.float32),
                   jax.ShapeDtypeStruct((_N, 32), jnp.float32)],
    )(lig_x, feat, emb_pad, lw, lb, rw, rb)


def _gather_body(tbl_hbm, idx_hbm, out_hbm, idx_v, rows_v, sem):
    wid = lax.axis_index("s") * _NC + lax.axis_index("c")
    n_it = jnp.where(wid < _ER_EXTRA, _ER_BASE + 1, _ER_BASE)

    def step(t, _):
        j = wid + t * _NW
        pltpu.sync_copy(idx_hbm.at[j], idx_v)
        pltpu.async_copy(tbl_hbm.at[idx_v], rows_v, sem).wait()
        pltpu.sync_copy(rows_v, out_hbm.at[pl.ds(j * _RPT, _RPT)])
        return _

    lax.fori_loop(0, n_it, step, 0)


def _run_gather(tbl, idx2d):
    mesh = plsc.VectorSubcoreMesh(core_axis_name="c", subcore_axis_name="s")
    f = pl.kernel(
        _gather_body,
        out_type=jax.ShapeDtypeStruct((_E, 32), jnp.float32),
        mesh=mesh,
        scratch_types=[
            pltpu.VMEM((_RPT,), jnp.int32),
            pltpu.VMEM((_RPT, 32), jnp.float32),
            pltpu.SemaphoreType.DMA,
        ],
        compiler_params=pltpu.CompilerParams(use_tc_tiling_on_sc=False),
    )
    return f(tbl, idx2d)


def _msg_body(ea_ref, hs_ref, w1_ref, b1_ref, w2_ref, b2_ref, t_ref, out_ref):
    u = jax.nn.relu(_dot16(ea_ref[...], w1_ref[...]) + b1_ref[...])
    w = _dot16(u, w2_ref[...]) + b2_ref[...]
    hsb = _dot16(hs_ref[...], t_ref[...])
    p = w.astype(jnp.bfloat16).astype(jnp.float32) * hsb
    q = p[:, 0:512] + p[:, 512:1024]
    q = q[:, 0:256] + q[:, 256:512]
    q = q[:, 0:128] + q[:, 128:256]
    out_ref[...] = (q[:, 0:32] + q[:, 32:64]) + (q[:, 64:96] + q[:, 96:128])


def _run_msg(ea, hs, w1, b1, w2, b2, t0):
    blk = 2000
    nb = _E // blk
    return pl.pallas_call(
        _msg_body,
        grid=(nb,),
        in_specs=[
            pl.BlockSpec((blk, 16), lambda e: (e, 0)),
            pl.BlockSpec((blk, 32), lambda e: (e, 0)),
            pl.BlockSpec((16, 128), lambda e: (0, 0)),
            pl.BlockSpec((1, 128), lambda e: (0, 0)),
            pl.BlockSpec((128, 1024), lambda e: (0, 0)),
            pl.BlockSpec((1, 1024), lambda e: (0, 0)),
            pl.BlockSpec((32, 1024), lambda e: (0, 0)),
        ],
        out_specs=pl.BlockSpec((blk, 32), lambda e: (e, 0)),
        out_shape=jax.ShapeDtypeStruct((_E, 32), jnp.float32),
    )(ea, hs, w1, b1, w2, b2, t0)


def _scatter_body(msg_hbm, dst_hbm, zero_hbm, out_hbm, acc_sh, idx_v, rows_v):
    c = lax.axis_index("c")
    s = lax.axis_index("s")
    wid = s * _NC + c

    @pl.when(s == 0)
    def _():
        pltpu.sync_copy(zero_hbm, acc_sh)

    plsc.subcore_barrier()

    n_it = jnp.where(wid < _ER_EXTRA, _ER_BASE + 1, _ER_BASE)

    def step(t, _):
        j = wid + t * _NW
        pltpu.sync_copy(dst_hbm.at[j], idx_v)
        pltpu.sync_copy(msg_hbm.at[pl.ds(j * _RPT, _RPT)], rows_v)
        pltpu.sync_copy(rows_v, acc_sh.at[idx_v], add=True)
        return _

    lax.fori_loop(0, n_it, step, 0)
    plsc.subcore_barrier()

    rows = _N // _NS
    pltpu.sync_copy(acc_sh.at[pl.ds(s * rows, rows)],
                    out_hbm.at[c, pl.ds(s * rows, rows)])


def _run_scatter(msg, dst2d, zeros):
    mesh = plsc.VectorSubcoreMesh(core_axis_name="c", subcore_axis_name="s")
    f = pl.kernel(
        _scatter_body,
        out_type=jax.ShapeDtypeStruct((_NC, _N, 32), jnp.float32),
        mesh=mesh,
        scratch_types=[
            pltpu.VMEM_SHARED((_N, 32), jnp.float32),
            pltpu.VMEM((_RPT,), jnp.int32),
            pltpu.VMEM((_RPT, 32), jnp.float32),
        ],
        compiler_params=pltpu.CompilerParams(use_tc_tiling_on_sc=False),
    )
    return f(msg, dst2d, zeros)


def _gru_body(parts_ref, nnb_ref, h_ref, wi_ref, bi_ref, wh_ref, bh_ref, out_ref):
    agg = parts_ref[0] + parts_ref[1] + nnb_ref[...]
    m = jax.nn.relu(agg)
    h = h_ref[...]
    gi = _dot16(m, wi_ref[...]) + bi_ref[...]
    gh = _dot16(h, wh_ref[...]) + bh_ref[...]
    r = jax.nn.sigmoid(gi[:, 0:32] + gh[:, 0:32])
    z = jax.nn.sigmoid(gi[:, 32:64] + gh[:, 32:64])
    n = jnp.tanh(gi[:, 64:96] + r * gh[:, 64:96])
    out_ref[...] = (1.0 - z) * n + z * h


def _run_gru(parts, nnb, h, wi, bi, wh, bh):
    blk = 2000
    nb = _N // blk
    return pl.pallas_call(
        _gru_body,
        grid=(nb,),
        in_specs=[
            pl.BlockSpec((2, blk, 32), lambda b: (0, b, 0)),
            pl.BlockSpec((1, 32), lambda b: (0, 0)),
            pl.BlockSpec((blk, 32), lambda b: (b, 0)),
            pl.BlockSpec((32, 96), lambda b: (0, 0)),
            pl.BlockSpec((1, 96), lambda b: (0, 0)),
            pl.BlockSpec((32, 96), lambda b: (0, 0)),
            pl.BlockSpec((1, 96), lambda b: (0, 0)),
        ],
        out_specs=pl.BlockSpec((blk, 32), lambda b: (b, 0)),
        out_shape=jax.ShapeDtypeStruct((_N, 32), jnp.float32),
    )(parts, nnb, h, wi, bi, wh, bh)


def _atn_body(lig_ref, rec_ref, wq_ref, bq_ref, wk_ref, bk_ref, wv_ref, bv_ref,
              wo_ref, bo_ref, cw_ref, cb_ref, rw_ref, rb_ref, lw_ref, lb_ref,
              out_ref):
    lig = lig_ref[...]
    rec = rec_ref[...]
    q = _dot16(lig, wq_ref[...]) + bq_ref[...]
    k = _dot16(rec, wk_ref[...]) + bk_ref[...]
    v = _dot16(rec, wv_ref[...]) + bv_ref[...]
    scores = lax.dot_general(q.astype(jnp.bfloat16), k.astype(jnp.bfloat16),
                             (((1,), (1,)), ((), ())),
                             preferred_element_type=jnp.float32) * (1.0 / (_DH ** 0.5))
    mx = jnp.max(scores, axis=1, keepdims=True)
    ex = jnp.exp(scores - mx)
    a = ex / jnp.sum(ex, axis=1, keepdims=True)
    av = _dot16(a, v)
    atn = _dot16(av, wo_ref[...]) + bo_ref[...]
    cat = jnp.concatenate([lig, atn], axis=1)
    lcomb = _dot16(cat, cw_ref[...]) + cb_ref[...]
    wr = jax.nn.sigmoid(_dot16(rec, rw_ref[...]) + rb_ref[...])
    hs_rec = jnp.sum(wr * rec, axis=0, keepdims=True)
    hm_rec = jnp.max(rec, axis=0, keepdims=True)
    wl = jax.nn.sigmoid(_dot16(lcomb, lw_ref[...]) + lb_ref[...])
    hs_lig = jnp.sum(wl * lcomb, axis=0, keepdims=True)
    hm_lig = jnp.max(lcomb, axis=0, keepdims=True)
    out_ref[0] = jnp.concatenate([hs_rec, hm_rec, hs_lig, hm_lig], axis=1)


def _run_atn(hid_lig, hid_rec, wqt, bq, wkt, bk, wvt, bv, wot, bo, cw, cb,
             rw, rb, lw, lb):
    small = lambda shape: pl.BlockSpec(shape, lambda g: tuple(0 for _ in shape))
    return pl.pallas_call(
        _atn_body,
        grid=(_G,),
        in_specs=[
            pl.BlockSpec((_NPG, 32), lambda g: (g, 0)),
            pl.BlockSpec((_NPG, 32), lambda g: (g, 0)),
            small((32, 32)), small((1, 32)),
            small((32, 32)), small((1, 32)),
            small((32, 32)), small((1, 32)),
            small((32, 32)), small((1, 32)),
            small((64, 32)), small((1, 32)),
            small((32, 1)), small((1, 1)),
            small((32, 1)), small((1, 1)),
        ],
        out_specs=pl.BlockSpec((1, 1, 128), lambda g: (g, 0, 0)),
        out_shape=jax.ShapeDtypeStruct((_G, 1, 128), jnp.float32),
    )(hid_lig, hid_rec, wqt, bq, wkt, bk, wvt, bv, wot, bo, cw, cb, rw, rb, lw, lb)


def _mlp_body(x_ref, w1_ref, b1_ref, w2_ref, b2_ref, wo_ref, bo_ref, out_ref):
    x = _dot16(x_ref[...], w1_ref[...]) + b1_ref[...]
    x = jnp.where(x > 0, x, 0.01 * x)
    x = _dot16(x, w2_ref[...]) + b2_ref[...]
    x = jnp.where(x > 0, x, 0.01 * x)
    out_ref[...] = _dot16(x, wo_ref[...]) + bo_ref[...]


def _run_mlp(x, w1, b1, w2, b2, wo, bo):
    return pl.pallas_call(
        _mlp_body,
        out_shape=jax.ShapeDtypeStruct((_G, 1), jnp.float32),
    )(x, w1, b1, w2, b2, wo, bo)


def kernel(lig_x, lig_edge_index, lig_edge_attr, rec_feat, rec_edge_index,
           rec_edge_attr, params):
    pg, pr = params['lig_gnn'], params['rec_gnn']
    f32 = jnp.float32

    emb_pad = jnp.zeros((32, 64), f32).at[:21].set(params['rec_embed'])
    h_lig, h_rec = _run_proj(lig_x, rec_feat, emb_pad,
                             pg['proj_W'], pg['proj_b'].reshape(1, 32),
                             pr['proj_W'], pr['proj_b'].reshape(1, 32))

    src_lig = lig_edge_index[0].reshape(_ER, _RPT)
    dst_lig = lig_edge_index[1].reshape(_ER, _RPT)
    src_rec = rec_edge_index[0].reshape(_ER, _RPT)
    dst_rec = rec_edge_index[1].reshape(_ER, _RPT)

    hsrc_lig = _run_gather(h_lig, src_lig)
    hsrc_rec = _run_gather(h_rec, src_rec)

    t0 = jnp.repeat(jnp.eye(32, dtype=f32), 32, axis=1)
    msg_lig = _run_msg(lig_edge_attr, hsrc_lig,
                       pg['eW1'], pg['eb1'].reshape(1, 128),
                       pg['eW2'], pg['eb2'].reshape(1, 1024), t0)
    msg_rec = _run_msg(rec_edge_attr, hsrc_rec,
                       pr['eW1'], pr['eb1'].reshape(1, 128),
                       pr['eW2'], pr['eb2'].reshape(1, 1024), t0)

    zeros = jnp.zeros((_N, 32), f32)
    parts_lig = _run_scatter(msg_lig, dst_lig, zeros)
    parts_rec = _run_scatter(msg_rec, dst_rec, zeros)

    hid_lig = _run_gru(parts_lig, pg['nn_bias'].reshape(1, 32), h_lig,
                       pg['gru_Wi'], pg['gru_bi'].reshape(1, 96),
                       pg['gru_Wh'], pg['gru_bh'].reshape(1, 96))
    hid_rec = _run_gru(parts_rec, pr['nn_bias'].reshape(1, 32), h_rec,
                       pr['gru_Wi'], pr['gru_bi'].reshape(1, 96),
                       pr['gru_Wh'], pr['gru_bh'].reshape(1, 96))

    a = params['atn']
    feats = _run_atn(
        hid_lig, hid_rec,
        a['Wq'].T, a['bq'].reshape(1, 32), a['Wk'].T, a['bk'].reshape(1, 32),
        a['Wv'].T, a['bv'].reshape(1, 32), a['Wo'].T, a['bo'].reshape(1, 32),
        params['comb_W'], params['comb_b'].reshape(1, 32),
        params['rec_ro_W'], params['rec_ro_b'].reshape(1, 1),
        params['lig_ro_W'], params['lig_ro_b'].reshape(1, 1),
    )

    m = params['mlp']
    return _run_mlp(feats.reshape(_G, 128),
                    m['W1'], m['b1'].reshape(1, 256),
                    m['W2'], m['b2'].reshape(1, 128),
                    m['Wo'], m['bo'].reshape(1, 1))
